# Initial kernel scaffold; baseline (speedup 1.0000x reference)
import jax
import jax.numpy as jnp
from jax.experimental import pallas as pl


def kernel(x, W, gamma, beta, edge_index, kernel_idx):
    raise NotImplementedError("write your pallas kernel here")


# speedup vs baseline: 1.1388x; 1.1388x over previous
import functools

import jax
import jax.numpy as jnp
from jax import lax
from jax.experimental import pallas as pl
from jax.experimental.pallas import tpu as pltpu
from jax.experimental.pallas import tpu_sc as plsc

_N = 10000
_E = 160000
_INC = 256
_OUTC = 256
_K = 27
_EPS = 1e-5

_HALF = 128
_HCOLS = _K * _HALF
_NS = 16
_EPT = _E // _NS
_CH = 80
_NCHUNK = _EPT // _CH
_NPAD = 10240
_RPT = _NPAD // _NS


def _mm_body(x_ref, w_ref, o_ref):
    o_ref[...] = jnp.dot(x_ref[...], w_ref[...],
                         preferred_element_type=jnp.float32)


def _matmul_half(x, w_half):
    BN, BC = 1000, 1152
    return pl.pallas_call(
        _mm_body,
        grid=(_N // BN, _HCOLS // BC),
        in_specs=[pl.BlockSpec((BN, _INC), lambda i, j: (i, 0)),
                  pl.BlockSpec((_INC, BC), lambda i, j: (0, j))],
        out_specs=pl.BlockSpec((BN, BC), lambda i, j: (i, j)),
        out_shape=jax.ShapeDtypeStruct((_N, _HCOLS), jnp.float32),
    )(x, w_half)


_sc_mesh = plsc.VectorSubcoreMesh(core_axis_name="c", subcore_axis_name="s")


@functools.partial(
    pl.kernel,
    out_type=(jax.ShapeDtypeStruct((_NPAD, _HALF), jnp.float32),
              jax.ShapeDtypeStruct((_NPAD, _HALF), jnp.float32)),
    mesh=_sc_mesh,
    scratch_types=[
        pltpu.VMEM((_CH,), jnp.int32),
        pltpu.VMEM((_CH,), jnp.int32),
        pltpu.VMEM((_CH,), jnp.int32),
        pltpu.VMEM((_CH,), jnp.int32),
        pltpu.VMEM((_CH, _HALF), jnp.float32),
        pltpu.VMEM_SHARED((_NPAD, _HALF), jnp.float32),
        pltpu.SemaphoreType.DMA,
    ],
)
def _sc_scatter(src_hbm, kidx_hbm, dst_hbm, hlo_hbm, hhi_hbm, zeros_hbm,
                outlo_hbm, outhi_hbm,
                src_v, kidx_v, addr_v, dst_v, rows_v, acc, sem):
    c = lax.axis_index("c")
    s = lax.axis_index("s")

    pltpu.sync_copy(zeros_hbm.at[pl.ds(s * _RPT, _RPT)],
                    acc.at[pl.ds(s * _RPT, _RPT)])
    plsc.subcore_barrier()

    def run(table_hbm, out_hbm):
        tile_base = s * _EPT

        def body(i, _):
            base = tile_base + i * _CH
            pltpu.sync_copy(src_hbm.at[pl.ds(base, _CH)], src_v)
            pltpu.sync_copy(kidx_hbm.at[pl.ds(base, _CH)], kidx_v)
            pltpu.sync_copy(dst_hbm.at[pl.ds(base, _CH)], dst_v)
            for j in range(_CH // 16):
                sl = pl.ds(16 * j, 16)
                addr_v[sl] = src_v[sl] * _K + kidx_v[sl]
            pltpu.async_copy(table_hbm.at[addr_v], rows_v, sem).wait()
            pltpu.sync_copy(rows_v, acc.at[dst_v], add=True)
            return 0

        lax.fori_loop(0, _NCHUNK, body, 0)
        plsc.subcore_barrier()
        pltpu.sync_copy(acc.at[pl.ds(s * _RPT, _RPT)],
                        out_hbm.at[pl.ds(s * _RPT, _RPT)])

    @pl.when(c == 0)
    def _():
        run(hlo_hbm, outlo_hbm)

    @pl.when(c == 1)
    def _():
        run(hhi_hbm, outhi_hbm)


def _stats_body(lo_ref, hi_ref, sum_ref, sq_ref, acc_s, acc_q):
    i = pl.program_id(0)

    @pl.when(i == 0)
    def _():
        acc_s[...] = jnp.zeros_like(acc_s)
        acc_q[...] = jnp.zeros_like(acc_q)

    v = jnp.concatenate([lo_ref[...], hi_ref[...]], axis=1)
    acc_s[...] += jnp.sum(v, axis=0, keepdims=True)
    acc_q[...] += jnp.sum(v * v, axis=0, keepdims=True)

    @pl.when(i == pl.num_programs(0) - 1)
    def _():
        sum_ref[...] = acc_s[...]
        sq_ref[...] = acc_q[...]


def _apply_body(lo_ref, hi_ref, sum_ref, sq_ref, g_ref, b_ref, o_ref):
    mu = sum_ref[...] / _N
    var = sq_ref[...] / _N - mu * mu
    scale = g_ref[...] * lax.rsqrt(var + _EPS)
    shift = b_ref[...] - mu * scale
    ylo = lo_ref[...] * scale[:, :_HALF] + shift[:, :_HALF]
    yhi = hi_ref[...] * scale[:, _HALF:] + shift[:, _HALF:]
    o_ref[:, :_HALF] = jnp.maximum(ylo, 0.0)
    o_ref[:, _HALF:] = jnp.maximum(yhi, 0.0)


def _batchnorm_relu(out_lo, out_hi, gamma, beta):
    BS = 2000
    g2 = gamma.reshape(1, _OUTC)
    b2 = beta.reshape(1, _OUTC)
    sums, sqs = pl.pallas_call(
        _stats_body,
        grid=(_N // BS,),
        in_specs=[pl.BlockSpec((BS, _HALF), lambda i: (i, 0)),
                  pl.BlockSpec((BS, _HALF), lambda i: (i, 0))],
        out_specs=[pl.BlockSpec((1, _OUTC), lambda i: (0, 0)),
                   pl.BlockSpec((1, _OUTC), lambda i: (0, 0))],
        out_shape=[jax.ShapeDtypeStruct((1, _OUTC), jnp.float32),
                   jax.ShapeDtypeStruct((1, _OUTC), jnp.float32)],
        scratch_shapes=[pltpu.VMEM((1, _OUTC), jnp.float32),
                        pltpu.VMEM((1, _OUTC), jnp.float32)],
    )(out_lo, out_hi)
    return pl.pallas_call(
        _apply_body,
        grid=(_N // BS,),
        in_specs=[pl.BlockSpec((BS, _HALF), lambda i: (i, 0)),
                  pl.BlockSpec((BS, _HALF), lambda i: (i, 0)),
                  pl.BlockSpec((1, _OUTC), lambda i: (0, 0)),
                  pl.BlockSpec((1, _OUTC), lambda i: (0, 0)),
                  pl.BlockSpec((1, _OUTC), lambda i: (0, 0)),
                  pl.BlockSpec((1, _OUTC), lambda i: (0, 0))],
        out_specs=pl.BlockSpec((BS, _OUTC), lambda i: (i, 0)),
        out_shape=jax.ShapeDtypeStruct((_N, _OUTC), jnp.float32),
    )(out_lo, out_hi, sums, sqs, g2, b2)


def kernel(x, W, gamma, beta, edge_index, kernel_idx):
    src = edge_index[0]
    dst = edge_index[1]
    wf = W.transpose(1, 0, 2).reshape(_INC, _K, 2, _HALF)
    wf = wf.transpose(0, 2, 1, 3).reshape(_INC, 2 * _HCOLS)
    h_lo = _matmul_half(x, wf[:, :_HCOLS]).reshape(_N * _K, _HALF)
    h_hi = _matmul_half(x, wf[:, _HCOLS:]).reshape(_N * _K, _HALF)
    zeros = jnp.zeros((_NPAD, _HALF), jnp.float32)
    out_lo, out_hi = _sc_scatter(src, kernel_idx, dst, h_lo, h_hi, zeros)
    return _batchnorm_relu(out_lo[:_N], out_hi[:_N], gamma, beta)


# speedup vs baseline: 2.2574x; 1.9822x over previous
import functools

import jax
import jax.numpy as jnp
from jax import lax
from jax.experimental import pallas as pl
from jax.experimental.pallas import tpu as pltpu
from jax.experimental.pallas import tpu_sc as plsc

_N = 10000
_E = 160000
_INC = 256
_OUTC = 256
_K = 27
_EPS = 1e-5

_HALF = 128
_HCOLS = _K * _HALF
_NS = 16
_EPT = _E // _NS
_CH = 40
_G = 10
_NSUP = _EPT // (_G * _CH)
_NPAD = 10240
_RPT = _NPAD // _NS


def _mm_body(x_ref, w_ref, o_ref):
    res = jnp.dot(x_ref[...], w_ref[...], preferred_element_type=jnp.float32)
    for kk in range(o_ref.shape[0]):
        o_ref[kk, :, :] = res[:, kk * _HALF:(kk + 1) * _HALF]


def _matmul_half(x, w_half):
    BN, KB = 1000, 9
    return pl.pallas_call(
        _mm_body,
        grid=(_N // BN, _K // KB),
        in_specs=[pl.BlockSpec((BN, _INC), lambda i, j: (i, 0)),
                  pl.BlockSpec((_INC, KB * _HALF), lambda i, j: (0, j))],
        out_specs=pl.BlockSpec((KB, BN, _HALF), lambda i, j: (j, i, 0)),
        out_shape=jax.ShapeDtypeStruct((_K, _N, _HALF), jnp.float32),
    )(x, w_half)


_sc_mesh = plsc.VectorSubcoreMesh(core_axis_name="c", subcore_axis_name="s")


@functools.partial(
    pl.kernel,
    out_type=(jax.ShapeDtypeStruct((_NPAD, _HALF), jnp.float32),
              jax.ShapeDtypeStruct((_NPAD, _HALF), jnp.float32)),
    mesh=_sc_mesh,
    scratch_types=[
        pltpu.VMEM((_G, 2, _CH), jnp.int32),
        pltpu.VMEM((2, _CH, _HALF), jnp.float32),
        pltpu.VMEM_SHARED((_NPAD, _HALF), jnp.float32),
        [pltpu.SemaphoreType.DMA] * 2,
    ],
)
def _sc_scatter(pack_hbm, hlo_hbm, hhi_hbm, zeros_hbm,
                outlo_hbm, outhi_hbm,
                ipack_v, rows_v, acc, sems):
    c = lax.axis_index("c")
    s = lax.axis_index("s")

    pltpu.sync_copy(zeros_hbm.at[pl.ds(s * _RPT, _RPT)],
                    acc.at[pl.ds(s * _RPT, _RPT)])
    plsc.subcore_barrier()

    def run(table_hbm, out_hbm):
        def body(i, _):
            pltpu.sync_copy(pack_hbm.at[s, i], ipack_v)
            for p in range(_G // 2):
                a, b = 2 * p, 2 * p + 1
                ga = pltpu.async_copy(table_hbm.at[ipack_v.at[a, 0]],
                                      rows_v.at[0], sems[0])
                gb = pltpu.async_copy(table_hbm.at[ipack_v.at[b, 0]],
                                      rows_v.at[1], sems[1])
                ga.wait()
                pltpu.sync_copy(rows_v.at[0], acc.at[ipack_v.at[a, 1]],
                                add=True)
                gb.wait()
                pltpu.sync_copy(rows_v.at[1], acc.at[ipack_v.at[b, 1]],
                                add=True)
            return 0

        lax.fori_loop(0, _NSUP, body, 0)
        plsc.subcore_barrier()
        pltpu.sync_copy(acc.at[pl.ds(s * _RPT, _RPT)],
                        out_hbm.at[pl.ds(s * _RPT, _RPT)])

    @pl.when(c == 0)
    def _():
        run(hlo_hbm, outlo_hbm)

    @pl.when(c == 1)
    def _():
        run(hhi_hbm, outhi_hbm)


def _stats_body(lo_ref, hi_ref, sum_ref, sq_ref, acc_s, acc_q):
    i = pl.program_id(0)

    @pl.when(i == 0)
    def _():
        acc_s[...] = jnp.zeros_like(acc_s)
        acc_q[...] = jnp.zeros_like(acc_q)

    v = jnp.concatenate([lo_ref[...], hi_ref[...]], axis=1)
    acc_s[...] += jnp.sum(v, axis=0, keepdims=True)
    acc_q[...] += jnp.sum(v * v, axis=0, keepdims=True)

    @pl.when(i == pl.num_programs(0) - 1)
    def _():
        sum_ref[...] = acc_s[...]
        sq_ref[...] = acc_q[...]


def _apply_body(lo_ref, hi_ref, sum_ref, sq_ref, g_ref, b_ref, o_ref):
    mu = sum_ref[...] / _N
    var = sq_ref[...] / _N - mu * mu
    scale = g_ref[...] * lax.rsqrt(var + _EPS)
    shift = b_ref[...] - mu * scale
    ylo = lo_ref[...] * scale[:, :_HALF] + shift[:, :_HALF]
    yhi = hi_ref[...] * scale[:, _HALF:] + shift[:, _HALF:]
    o_ref[:, :_HALF] = jnp.maximum(ylo, 0.0)
    o_ref[:, _HALF:] = jnp.maximum(yhi, 0.0)


def _batchnorm_relu(out_lo, out_hi, gamma, beta):
    BS = 2000
    g2 = gamma.reshape(1, _OUTC)
    b2 = beta.reshape(1, _OUTC)
    sums, sqs = pl.pallas_call(
        _stats_body,
        grid=(_N // BS,),
        in_specs=[pl.BlockSpec((BS, _HALF), lambda i: (i, 0)),
                  pl.BlockSpec((BS, _HALF), lambda i: (i, 0))],
        out_specs=[pl.BlockSpec((1, _OUTC), lambda i: (0, 0)),
                   pl.BlockSpec((1, _OUTC), lambda i: (0, 0))],
        out_shape=[jax.ShapeDtypeStruct((1, _OUTC), jnp.float32),
                   jax.ShapeDtypeStruct((1, _OUTC), jnp.float32)],
        scratch_shapes=[pltpu.VMEM((1, _OUTC), jnp.float32),
                        pltpu.VMEM((1, _OUTC), jnp.float32)],
    )(out_lo, out_hi)
    return pl.pallas_call(
        _apply_body,
        grid=(_N // BS,),
        in_specs=[pl.BlockSpec((BS, _HALF), lambda i: (i, 0)),
                  pl.BlockSpec((BS, _HALF), lambda i: (i, 0)),
                  pl.BlockSpec((1, _OUTC), lambda i: (0, 0)),
                  pl.BlockSpec((1, _OUTC), lambda i: (0, 0)),
                  pl.BlockSpec((1, _OUTC), lambda i: (0, 0)),
                  pl.BlockSpec((1, _OUTC), lambda i: (0, 0))],
        out_specs=pl.BlockSpec((BS, _OUTC), lambda i: (i, 0)),
        out_shape=jax.ShapeDtypeStruct((_N, _OUTC), jnp.float32),
    )(out_lo, out_hi, sums, sqs, g2, b2)


def kernel(x, W, gamma, beta, edge_index, kernel_idx):
    addr = kernel_idx * _N + edge_index[0]
    addr4 = addr.reshape(_NS, _NSUP, _G, _CH)
    dst4 = edge_index[1].reshape(_NS, _NSUP, _G, _CH)
    pack = jnp.stack([addr4, dst4], axis=3)
    wf = W.transpose(1, 0, 2).reshape(_INC, _K, 2, _HALF)
    wf = wf.transpose(0, 2, 1, 3).reshape(_INC, 2 * _HCOLS)
    h_lo = _matmul_half(x, wf[:, :_HCOLS]).reshape(_K * _N, _HALF)
    h_hi = _matmul_half(x, wf[:, _HCOLS:]).reshape(_K * _N, _HALF)
    zeros = jnp.zeros((_NPAD, _HALF), jnp.float32)
    out_lo, out_hi = _sc_scatter(pack, h_lo, h_hi, zeros)
    return _batchnorm_relu(out_lo, out_hi, gamma, beta)


# speedup vs baseline: 2.3034x; 1.0204x over previous
import functools

import jax
import jax.numpy as jnp
from jax import lax
from jax.experimental import pallas as pl
from jax.experimental.pallas import tpu as pltpu
from jax.experimental.pallas import tpu_sc as plsc

_N = 10000
_E = 160000
_INC = 256
_OUTC = 256
_K = 27
_EPS = 1e-5

_HALF = 128
_HCOLS = _K * _HALF
_NS = 16
_EPT = _E // _NS
_CH = 16
_G = 25
_NSUP = _EPT // (_G * _CH)
_NBUF = 5
_NPAD = 10240
_RPT = _NPAD // _NS


def _mm_body(x_ref, w_ref, o_ref):
    res = jnp.dot(x_ref[...], w_ref[...], preferred_element_type=jnp.float32)
    for kk in range(o_ref.shape[0]):
        o_ref[kk, :, :] = res[:, kk * _HALF:(kk + 1) * _HALF]


def _matmul_half(x, w_half):
    BN, KB = 1000, 9
    return pl.pallas_call(
        _mm_body,
        grid=(_N // BN, _K // KB),
        in_specs=[pl.BlockSpec((BN, _INC), lambda i, j: (i, 0)),
                  pl.BlockSpec((_INC, KB * _HALF), lambda i, j: (0, j))],
        out_specs=pl.BlockSpec((KB, BN, _HALF), lambda i, j: (j, i, 0)),
        out_shape=jax.ShapeDtypeStruct((_K, _N, _HALF), jnp.float32),
    )(x, w_half)


_sc_mesh = plsc.VectorSubcoreMesh(core_axis_name="c", subcore_axis_name="s")


@functools.partial(
    pl.kernel,
    out_type=(jax.ShapeDtypeStruct((_NPAD, _HALF), jnp.float32),
              jax.ShapeDtypeStruct((_NPAD, _HALF), jnp.float32)),
    mesh=_sc_mesh,
    scratch_types=[
        pltpu.VMEM((_G, 2, _CH), jnp.int32),
        pltpu.VMEM((_NBUF, _CH, _HALF), jnp.float32),
        pltpu.VMEM_SHARED((_NPAD, _HALF), jnp.float32),
        [pltpu.SemaphoreType.DMA] * _NBUF,
        [pltpu.SemaphoreType.DMA] * _NBUF,
    ],
)
def _sc_scatter(pack_hbm, hlo_hbm, hhi_hbm, zeros_hbm,
                outlo_hbm, outhi_hbm,
                ipack_v, rows_v, acc, gsems, ssems):
    c = lax.axis_index("c")
    s = lax.axis_index("s")

    pltpu.sync_copy(zeros_hbm.at[pl.ds(s * _RPT, _RPT)],
                    acc.at[pl.ds(s * _RPT, _RPT)])
    plsc.subcore_barrier()

    def run(table_hbm, out_hbm):
        def sup_body(si, _):
            pltpu.sync_copy(pack_hbm.at[s, si], ipack_v)

            def grp_body(k, _):
                base = k * _NBUF
                gs = []
                for b in range(_NBUF):
                    @pl.when(k > 0)
                    def _(b=b):
                        pltpu.make_async_copy(
                            rows_v.at[b], acc.at[ipack_v.at[base + b, 1]],
                            ssems[b]).wait()
                    gs.append(pltpu.async_copy(
                        table_hbm.at[ipack_v.at[base + b, 0]],
                        rows_v.at[b], gsems[b]))
                for b in range(_NBUF):
                    gs[b].wait()
                    pltpu.async_copy(rows_v.at[b],
                                     acc.at[ipack_v.at[base + b, 1]],
                                     ssems[b], add=True)
                return 0

            lax.fori_loop(0, _G // _NBUF, grp_body, 0)
            last = (_G // _NBUF - 1) * _NBUF
            for b in range(_NBUF):
                pltpu.make_async_copy(rows_v.at[b],
                                      acc.at[ipack_v.at[last + b, 1]],
                                      ssems[b]).wait()
            return 0

        lax.fori_loop(0, _NSUP, sup_body, 0)
        plsc.subcore_barrier()
        pltpu.sync_copy(acc.at[pl.ds(s * _RPT, _RPT)],
                        out_hbm.at[pl.ds(s * _RPT, _RPT)])

    @pl.when(c == 0)
    def _():
        run(hlo_hbm, outlo_hbm)

    @pl.when(c == 1)
    def _():
        run(hhi_hbm, outhi_hbm)


def _stats_body(lo_ref, hi_ref, sum_ref, sq_ref, acc_s, acc_q):
    i = pl.program_id(0)

    @pl.when(i == 0)
    def _():
        acc_s[...] = jnp.zeros_like(acc_s)
        acc_q[...] = jnp.zeros_like(acc_q)

    v = jnp.concatenate([lo_ref[...], hi_ref[...]], axis=1)
    acc_s[...] += jnp.sum(v, axis=0, keepdims=True)
    acc_q[...] += jnp.sum(v * v, axis=0, keepdims=True)

    @pl.when(i == pl.num_programs(0) - 1)
    def _():
        sum_ref[...] = acc_s[...]
        sq_ref[...] = acc_q[...]


def _apply_body(lo_ref, hi_ref, sum_ref, sq_ref, g_ref, b_ref, o_ref):
    mu = sum_ref[...] / _N
    var = sq_ref[...] / _N - mu * mu
    scale = g_ref[...] * lax.rsqrt(var + _EPS)
    shift = b_ref[...] - mu * scale
    ylo = lo_ref[...] * scale[:, :_HALF] + shift[:, :_HALF]
    yhi = hi_ref[...] * scale[:, _HALF:] + shift[:, _HALF:]
    o_ref[:, :_HALF] = jnp.maximum(ylo, 0.0)
    o_ref[:, _HALF:] = jnp.maximum(yhi, 0.0)


def _batchnorm_relu(out_lo, out_hi, gamma, beta):
    BS = 2000
    g2 = gamma.reshape(1, _OUTC)
    b2 = beta.reshape(1, _OUTC)
    sums, sqs = pl.pallas_call(
        _stats_body,
        grid=(_N // BS,),
        in_specs=[pl.BlockSpec((BS, _HALF), lambda i: (i, 0)),
                  pl.BlockSpec((BS, _HALF), lambda i: (i, 0))],
        out_specs=[pl.BlockSpec((1, _OUTC), lambda i: (0, 0)),
                   pl.BlockSpec((1, _OUTC), lambda i: (0, 0))],
        out_shape=[jax.ShapeDtypeStruct((1, _OUTC), jnp.float32),
                   jax.ShapeDtypeStruct((1, _OUTC), jnp.float32)],
        scratch_shapes=[pltpu.VMEM((1, _OUTC), jnp.float32),
                        pltpu.VMEM((1, _OUTC), jnp.float32)],
    )(out_lo, out_hi)
    return pl.pallas_call(
        _apply_body,
        grid=(_N // BS,),
        in_specs=[pl.BlockSpec((BS, _HALF), lambda i: (i, 0)),
                  pl.BlockSpec((BS, _HALF), lambda i: (i, 0)),
                  pl.BlockSpec((1, _OUTC), lambda i: (0, 0)),
                  pl.BlockSpec((1, _OUTC), lambda i: (0, 0)),
                  pl.BlockSpec((1, _OUTC), lambda i: (0, 0)),
                  pl.BlockSpec((1, _OUTC), lambda i: (0, 0))],
        out_specs=pl.BlockSpec((BS, _OUTC), lambda i: (i, 0)),
        out_shape=jax.ShapeDtypeStruct((_N, _OUTC), jnp.float32),
    )(out_lo, out_hi, sums, sqs, g2, b2)


def kernel(x, W, gamma, beta, edge_index, kernel_idx):
    addr = kernel_idx * _N + edge_index[0]
    addr4 = addr.reshape(_NS, _NSUP, _G, _CH)
    dst4 = edge_index[1].reshape(_NS, _NSUP, _G, _CH)
    pack = jnp.stack([addr4, dst4], axis=3)
    del addr4, dst4
    wf = W.transpose(1, 0, 2).reshape(_INC, _K, 2, _HALF)
    wf = wf.transpose(0, 2, 1, 3).reshape(_INC, 2 * _HCOLS)
    h_lo = _matmul_half(x, wf[:, :_HCOLS]).reshape(_K * _N, _HALF)
    h_hi = _matmul_half(x, wf[:, _HCOLS:]).reshape(_K * _N, _HALF)
    zeros = jnp.zeros((_NPAD, _HALF), jnp.float32)
    out_lo, out_hi = _sc_scatter(pack, h_lo, h_hi, zeros)
    return _batchnorm_relu(out_lo, out_hi, gamma, beta)


# speedup vs baseline: 2.3723x; 1.0299x over previous
import functools

import jax
import jax.numpy as jnp
from jax import lax
from jax.experimental import pallas as pl
from jax.experimental.pallas import tpu as pltpu
from jax.experimental.pallas import tpu_sc as plsc

_N = 10000
_E = 160000
_INC = 256
_OUTC = 256
_K = 27
_EPS = 1e-5

_HALF = 128
_HCOLS = _K * _HALF
_NS = 16
_EPT = _E // _NS
_CH = 16
_G = 25
_NSUP = _EPT // (_G * _CH)
_NBUF = 5
_NPAD = 10240
_RPT = _NPAD // _NS


def _mm_body(x_ref, w_ref, o_ref):
    res = jnp.dot(x_ref[...], w_ref[...], preferred_element_type=jnp.float32)
    for kk in range(o_ref.shape[0]):
        o_ref[kk, :, :] = res[:, kk * _HALF:(kk + 1) * _HALF]


def _matmul_half(x, w_half):
    BN, KB = 1000, 9
    return pl.pallas_call(
        _mm_body,
        grid=(_N // BN, _K // KB),
        in_specs=[pl.BlockSpec((BN, _INC), lambda i, j: (i, 0)),
                  pl.BlockSpec((_INC, KB * _HALF), lambda i, j: (0, j))],
        out_specs=pl.BlockSpec((KB, BN, _HALF), lambda i, j: (j, i, 0)),
        out_shape=jax.ShapeDtypeStruct((_K, _N, _HALF), jnp.float32),
    )(x.astype(jnp.bfloat16), w_half.astype(jnp.bfloat16))


_sc_mesh = plsc.VectorSubcoreMesh(core_axis_name="c", subcore_axis_name="s")


@functools.partial(
    pl.kernel,
    out_type=(jax.ShapeDtypeStruct((_NPAD, _HALF), jnp.float32),
              jax.ShapeDtypeStruct((_NPAD, _HALF), jnp.float32)),
    mesh=_sc_mesh,
    scratch_types=[
        pltpu.VMEM((_G, 2, _CH), jnp.int32),
        pltpu.VMEM((_NBUF, _CH, _HALF), jnp.float32),
        pltpu.VMEM_SHARED((_NPAD, _HALF), jnp.float32),
        [pltpu.SemaphoreType.DMA] * _NBUF,
        [pltpu.SemaphoreType.DMA] * _NBUF,
    ],
)
def _sc_scatter(pack_hbm, hlo_hbm, hhi_hbm, zeros_hbm,
                outlo_hbm, outhi_hbm,
                ipack_v, rows_v, acc, gsems, ssems):
    c = lax.axis_index("c")
    s = lax.axis_index("s")

    pltpu.sync_copy(zeros_hbm.at[pl.ds(s * _RPT, _RPT)],
                    acc.at[pl.ds(s * _RPT, _RPT)])
    plsc.subcore_barrier()

    def run(table_hbm, out_hbm):
        def sup_body(si, _):
            pltpu.sync_copy(pack_hbm.at[s, si], ipack_v)

            def grp_body(k, _):
                base = k * _NBUF
                gs = []
                for b in range(_NBUF):
                    @pl.when(k > 0)
                    def _(b=b):
                        pltpu.make_async_copy(
                            rows_v.at[b], acc.at[ipack_v.at[base + b, 1]],
                            ssems[b]).wait()
                    gs.append(pltpu.async_copy(
                        table_hbm.at[ipack_v.at[base + b, 0]],
                        rows_v.at[b], gsems[b]))
                for b in range(_NBUF):
                    gs[b].wait()
                    pltpu.async_copy(rows_v.at[b],
                                     acc.at[ipack_v.at[base + b, 1]],
                                     ssems[b], add=True)
                return 0

            lax.fori_loop(0, _G // _NBUF, grp_body, 0)
            last = (_G // _NBUF - 1) * _NBUF
            for b in range(_NBUF):
                pltpu.make_async_copy(rows_v.at[b],
                                      acc.at[ipack_v.at[last + b, 1]],
                                      ssems[b]).wait()
            return 0

        lax.fori_loop(0, _NSUP, sup_body, 0)
        plsc.subcore_barrier()
        pltpu.sync_copy(acc.at[pl.ds(s * _RPT, _RPT)],
                        out_hbm.at[pl.ds(s * _RPT, _RPT)])

    @pl.when(c == 0)
    def _():
        run(hlo_hbm, outlo_hbm)

    @pl.when(c == 1)
    def _():
        run(hhi_hbm, outhi_hbm)


def _stats_body(lo_ref, hi_ref, sum_ref, sq_ref, acc_s, acc_q):
    i = pl.program_id(0)

    @pl.when(i == 0)
    def _():
        acc_s[...] = jnp.zeros_like(acc_s)
        acc_q[...] = jnp.zeros_like(acc_q)

    v = jnp.concatenate([lo_ref[...], hi_ref[...]], axis=1)
    acc_s[...] += jnp.sum(v, axis=0, keepdims=True)
    acc_q[...] += jnp.sum(v * v, axis=0, keepdims=True)

    @pl.when(i == pl.num_programs(0) - 1)
    def _():
        sum_ref[...] = acc_s[...]
        sq_ref[...] = acc_q[...]


def _apply_body(lo_ref, hi_ref, sum_ref, sq_ref, g_ref, b_ref, o_ref):
    mu = sum_ref[...] / _N
    var = sq_ref[...] / _N - mu * mu
    scale = g_ref[...] * lax.rsqrt(var + _EPS)
    shift = b_ref[...] - mu * scale
    ylo = lo_ref[...] * scale[:, :_HALF] + shift[:, :_HALF]
    yhi = hi_ref[...] * scale[:, _HALF:] + shift[:, _HALF:]
    o_ref[:, :_HALF] = jnp.maximum(ylo, 0.0)
    o_ref[:, _HALF:] = jnp.maximum(yhi, 0.0)


def _batchnorm_relu(out_lo, out_hi, gamma, beta):
    BS = 2000
    g2 = gamma.reshape(1, _OUTC)
    b2 = beta.reshape(1, _OUTC)
    sums, sqs = pl.pallas_call(
        _stats_body,
        grid=(_N // BS,),
        in_specs=[pl.BlockSpec((BS, _HALF), lambda i: (i, 0)),
                  pl.BlockSpec((BS, _HALF), lambda i: (i, 0))],
        out_specs=[pl.BlockSpec((1, _OUTC), lambda i: (0, 0)),
                   pl.BlockSpec((1, _OUTC), lambda i: (0, 0))],
        out_shape=[jax.ShapeDtypeStruct((1, _OUTC), jnp.float32),
                   jax.ShapeDtypeStruct((1, _OUTC), jnp.float32)],
        scratch_shapes=[pltpu.VMEM((1, _OUTC), jnp.float32),
                        pltpu.VMEM((1, _OUTC), jnp.float32)],
    )(out_lo, out_hi)
    return pl.pallas_call(
        _apply_body,
        grid=(_N // BS,),
        in_specs=[pl.BlockSpec((BS, _HALF), lambda i: (i, 0)),
                  pl.BlockSpec((BS, _HALF), lambda i: (i, 0)),
                  pl.BlockSpec((1, _OUTC), lambda i: (0, 0)),
                  pl.BlockSpec((1, _OUTC), lambda i: (0, 0)),
                  pl.BlockSpec((1, _OUTC), lambda i: (0, 0)),
                  pl.BlockSpec((1, _OUTC), lambda i: (0, 0))],
        out_specs=pl.BlockSpec((BS, _OUTC), lambda i: (i, 0)),
        out_shape=jax.ShapeDtypeStruct((_N, _OUTC), jnp.float32),
    )(out_lo, out_hi, sums, sqs, g2, b2)


def kernel(x, W, gamma, beta, edge_index, kernel_idx):
    addr = kernel_idx * _N + edge_index[0]
    addr4 = addr.reshape(_NS, _NSUP, _G, _CH)
    dst4 = edge_index[1].reshape(_NS, _NSUP, _G, _CH)
    pack = jnp.stack([addr4, dst4], axis=3)
    del addr4, dst4
    wf = W.transpose(1, 0, 2).reshape(_INC, _K, 2, _HALF)
    wf = wf.transpose(0, 2, 1, 3).reshape(_INC, 2 * _HCOLS)
    h_lo = _matmul_half(x, wf[:, :_HCOLS]).reshape(_K * _N, _HALF)
    h_hi = _matmul_half(x, wf[:, _HCOLS:]).reshape(_K * _N, _HALF)
    zeros = jnp.zeros((_NPAD, _HALF), jnp.float32)
    out_lo, out_hi = _sc_scatter(pack, h_lo, h_hi, zeros)
    return _batchnorm_relu(out_lo, out_hi, gamma, beta)


# speedup vs baseline: 2.3766x; 1.0018x over previous
import functools

import jax
import jax.numpy as jnp
from jax import lax
from jax.experimental import pallas as pl
from jax.experimental.pallas import tpu as pltpu
from jax.experimental.pallas import tpu_sc as plsc

_N = 10000
_E = 160000
_INC = 256
_OUTC = 256
_K = 27
_EPS = 1e-5

_HALF = 128
_HCOLS = _K * _HALF
_NS = 16
_EPT = _E // _NS
_CH = 16
_G = 25
_NSUP = _EPT // (_G * _CH)
_NBUF = 5
_NPAD = 10240
_RPT = _NPAD // _NS


def _mm_body(x_ref, w_ref, o_ref):
    res = jnp.dot(x_ref[...], w_ref[...], preferred_element_type=jnp.float32)
    for kk in range(o_ref.shape[0]):
        o_ref[kk, :, :] = res[:, kk * _HALF:(kk + 1) * _HALF]


def _matmul_half(x, w_half):
    BN, KB = 1000, 9
    return pl.pallas_call(
        _mm_body,
        grid=(_N // BN, _K // KB),
        in_specs=[pl.BlockSpec((BN, _INC), lambda i, j: (i, 0)),
                  pl.BlockSpec((_INC, KB * _HALF), lambda i, j: (0, j))],
        out_specs=pl.BlockSpec((KB, BN, _HALF), lambda i, j: (j, i, 0)),
        out_shape=jax.ShapeDtypeStruct((_K, _N, _HALF), jnp.float32),
    )(x, w_half)


_sc_mesh = plsc.VectorSubcoreMesh(core_axis_name="c", subcore_axis_name="s")


@functools.partial(
    pl.kernel,
    out_type=(jax.ShapeDtypeStruct((_NPAD, _HALF), jnp.float32),
              jax.ShapeDtypeStruct((_NPAD, _HALF), jnp.float32)),
    mesh=_sc_mesh,
    scratch_types=[
        pltpu.VMEM((_G, 2, _CH), jnp.int32),
        pltpu.VMEM((_NBUF, _CH, _HALF), jnp.float32),
        pltpu.VMEM_SHARED((_NPAD, _HALF), jnp.float32),
        [pltpu.SemaphoreType.DMA] * _NBUF,
        [pltpu.SemaphoreType.DMA] * _NBUF,
    ],
)
def _sc_scatter(pack_hbm, hlo_hbm, hhi_hbm, zeros_hbm,
                outlo_hbm, outhi_hbm,
                ipack_v, rows_v, acc, gsems, ssems):
    c = lax.axis_index("c")
    s = lax.axis_index("s")

    pltpu.sync_copy(zeros_hbm.at[pl.ds(s * _RPT, _RPT)],
                    acc.at[pl.ds(s * _RPT, _RPT)])
    plsc.subcore_barrier()

    def run(table_hbm, out_hbm):
        def sup_body(si, _):
            pltpu.sync_copy(pack_hbm.at[s, si], ipack_v)

            def grp_body(k, _):
                base = k * _NBUF
                gs = []
                for b in range(_NBUF):
                    @pl.when(k > 0)
                    def _(b=b):
                        pltpu.make_async_copy(
                            rows_v.at[b], acc.at[ipack_v.at[base + b, 1]],
                            ssems[b]).wait()
                    gs.append(pltpu.async_copy(
                        table_hbm.at[ipack_v.at[base + b, 0]],
                        rows_v.at[b], gsems[b]))
                for b in range(_NBUF):
                    gs[b].wait()
                    pltpu.async_copy(rows_v.at[b],
                                     acc.at[ipack_v.at[base + b, 1]],
                                     ssems[b], add=True)
                return 0

            lax.fori_loop(0, _G // _NBUF, grp_body, 0)
            last = (_G // _NBUF - 1) * _NBUF
            for b in range(_NBUF):
                pltpu.make_async_copy(rows_v.at[b],
                                      acc.at[ipack_v.at[last + b, 1]],
                                      ssems[b]).wait()
            return 0

        lax.fori_loop(0, _NSUP, sup_body, 0)
        plsc.subcore_barrier()
        pltpu.sync_copy(acc.at[pl.ds(s * _RPT, _RPT)],
                        out_hbm.at[pl.ds(s * _RPT, _RPT)])

    @pl.when(c == 0)
    def _():
        run(hlo_hbm, outlo_hbm)

    @pl.when(c == 1)
    def _():
        run(hhi_hbm, outhi_hbm)


def _stats_body(lo_ref, hi_ref, sum_ref, sq_ref, acc_s, acc_q):
    i = pl.program_id(0)

    @pl.when(i == 0)
    def _():
        acc_s[...] = jnp.zeros_like(acc_s)
        acc_q[...] = jnp.zeros_like(acc_q)

    v = jnp.concatenate([lo_ref[...], hi_ref[...]], axis=1)
    acc_s[...] += jnp.sum(v, axis=0, keepdims=True)
    acc_q[...] += jnp.sum(v * v, axis=0, keepdims=True)

    @pl.when(i == pl.num_programs(0) - 1)
    def _():
        sum_ref[...] = acc_s[...]
        sq_ref[...] = acc_q[...]


def _apply_body(lo_ref, hi_ref, sum_ref, sq_ref, g_ref, b_ref, o_ref):
    mu = sum_ref[...] / _N
    var = sq_ref[...] / _N - mu * mu
    scale = g_ref[...] * lax.rsqrt(var + _EPS)
    shift = b_ref[...] - mu * scale
    ylo = lo_ref[...] * scale[:, :_HALF] + shift[:, :_HALF]
    yhi = hi_ref[...] * scale[:, _HALF:] + shift[:, _HALF:]
    o_ref[:, :_HALF] = jnp.maximum(ylo, 0.0)
    o_ref[:, _HALF:] = jnp.maximum(yhi, 0.0)


def _batchnorm_relu(out_lo, out_hi, gamma, beta):
    BS = 2000
    g2 = gamma.reshape(1, _OUTC)
    b2 = beta.reshape(1, _OUTC)
    sums, sqs = pl.pallas_call(
        _stats_body,
        grid=(_N // BS,),
        in_specs=[pl.BlockSpec((BS, _HALF), lambda i: (i, 0)),
                  pl.BlockSpec((BS, _HALF), lambda i: (i, 0))],
        out_specs=[pl.BlockSpec((1, _OUTC), lambda i: (0, 0)),
                   pl.BlockSpec((1, _OUTC), lambda i: (0, 0))],
        out_shape=[jax.ShapeDtypeStruct((1, _OUTC), jnp.float32),
                   jax.ShapeDtypeStruct((1, _OUTC), jnp.float32)],
        scratch_shapes=[pltpu.VMEM((1, _OUTC), jnp.float32),
                        pltpu.VMEM((1, _OUTC), jnp.float32)],
    )(out_lo, out_hi)
    return pl.pallas_call(
        _apply_body,
        grid=(_N // BS,),
        in_specs=[pl.BlockSpec((BS, _HALF), lambda i: (i, 0)),
                  pl.BlockSpec((BS, _HALF), lambda i: (i, 0)),
                  pl.BlockSpec((1, _OUTC), lambda i: (0, 0)),
                  pl.BlockSpec((1, _OUTC), lambda i: (0, 0)),
                  pl.BlockSpec((1, _OUTC), lambda i: (0, 0)),
                  pl.BlockSpec((1, _OUTC), lambda i: (0, 0))],
        out_specs=pl.BlockSpec((BS, _OUTC), lambda i: (i, 0)),
        out_shape=jax.ShapeDtypeStruct((_N, _OUTC), jnp.float32),
    )(out_lo, out_hi, sums, sqs, g2, b2)


def kernel(x, W, gamma, beta, edge_index, kernel_idx):
    addr = kernel_idx * _N + edge_index[0]
    addr4 = addr.reshape(_NS, _NSUP, _G, _CH)
    dst4 = edge_index[1].reshape(_NS, _NSUP, _G, _CH)
    pack = jnp.stack([addr4, dst4], axis=3)
    del addr4, dst4
    wf = W.astype(jnp.bfloat16).transpose(1, 0, 2).reshape(_INC, _K, 2, _HALF)
    wf = wf.transpose(0, 2, 1, 3).reshape(_INC, 2 * _HCOLS)
    xb = x.astype(jnp.bfloat16)
    h_lo = _matmul_half(xb, wf[:, :_HCOLS]).reshape(_K * _N, _HALF)
    h_hi = _matmul_half(xb, wf[:, _HCOLS:]).reshape(_K * _N, _HALF)
    zeros = jnp.zeros((_NPAD, _HALF), jnp.float32)
    out_lo, out_hi = _sc_scatter(pack, h_lo, h_hi, zeros)
    return _batchnorm_relu(out_lo, out_hi, gamma, beta)
